# baseline (speedup 1.0000x reference)
import functools

import jax
import jax.numpy as jnp
from jax import lax
from jax._src.pallas import mpmd
from jax.experimental import pallas as pl
from jax.experimental.pallas import tpu as pltpu
from jax.experimental.pallas import tpu_sc as plsc

_NUM_CORES = 2
_NUM_SUBCORES = 16
_LANES = 128
_ALIGN = 1024
_PIPE_LAG = 8
_Q_SPLIT = 256
_SCS_REGION_ROWS = 392


def _tec_body(q_len, v_len, dim, max_pos, region_rows,
              big_hbm, out_hbm, table_v, table_sh, sem_tec, sem_scs):
  del table_sh, sem_scs
  cid = lax.axis_index("c")
  sid = lax.axis_index("s")

  pltpu.sync_copy(big_hbm.at[pl.ds(sid * region_rows, region_rows), :],
                  table_v)

  row_rows = v_len * dim // _LANES
  pad = lax.rem((16 - sid) * 64, _ALIGN)
  q_lo = lax.rem(max_pos - sid, 16)

  copies = []
  for k in range(_Q_SPLIT // 16):
    q_row = q_lo + 16 * k
    s = (max_pos - q_row) * dim
    src_row = lax.div(pad + s, _LANES)
    for bb in range(2):
      r = (cid * 2 + bb) * q_len + q_row
      c = pltpu.make_async_copy(
          table_v.at[pl.ds(src_row, row_rows), :],
          out_hbm.at[pl.ds(r * row_rows, row_rows), :],
          sem_tec,
      )
      c.start()
      copies.append(c)
      if len(copies) > _PIPE_LAG:
        copies[len(copies) - 1 - _PIPE_LAG].wait()
  for c in copies[-_PIPE_LAG:]:
    c.wait()


def _scs_body(q_len, v_len, dim, max_pos, region_rows,
              big_hbm, out_hbm, table_v, table_sh, sem_tec, sem_scs):
  del table_v, sem_tec
  cid = lax.axis_index("c")

  for j in range(16):
    pltpu.sync_copy(big_hbm.at[pl.ds(j * region_rows, _SCS_REGION_ROWS), :],
                    table_sh.at[pl.ds(j * _SCS_REGION_ROWS,
                                      _SCS_REGION_ROWS), :])

  row_rows = v_len * dim // _LANES
  n_q = q_len - _Q_SPLIT

  def step(i, carry):
    q_row = _Q_SPLIT + lax.div(i, 2)
    bb = lax.rem(i, 2)
    s = (max_pos - q_row) * dim
    j = lax.rem(lax.div(s, 64), 16)
    pad = lax.rem((16 - j) * 64, _ALIGN)
    src_row = j * _SCS_REGION_ROWS + lax.div(pad + s, _LANES)
    r = (cid * 2 + bb) * q_len + q_row
    pltpu.make_async_copy(
        table_sh.at[pl.ds(src_row, row_rows), :],
        out_hbm.at[pl.ds(r * row_rows, row_rows), :],
        sem_scs,
    ).start()

    @pl.when(i >= _PIPE_LAG)
    def _drain():
      pltpu.make_async_copy(
          table_sh.at[pl.ds(0, row_rows), :],
          out_hbm.at[pl.ds(0, row_rows), :],
          sem_scs,
      ).wait()

    return carry

  lax.fori_loop(0, 2 * n_q, step, 0)
  for _ in range(_PIPE_LAG):
    pltpu.make_async_copy(
        table_sh.at[pl.ds(0, row_rows), :],
        out_hbm.at[pl.ds(0, row_rows), :],
        sem_scs,
    ).wait()


def kernel(q, v, embeddings):
  batch, q_len = q.shape[0], q.shape[1]
  v_len = v.shape[1]
  table_rows, dim = embeddings.shape
  max_pos = (table_rows - 1) // 2

  assert batch == 2 * _NUM_CORES and q_len % _NUM_SUBCORES == 0
  assert (v_len * dim) % _LANES == 0

  table_words = table_rows * dim
  region_words = -(-(960 + table_words) // _ALIGN) * _ALIGN
  region_rows = region_words // _LANES

  flat = embeddings.reshape(-1)
  big = jnp.zeros((16 * region_words,), jnp.float32)
  for j in range(16):
    pad = (16 - j) * 64 % _ALIGN
    big = lax.dynamic_update_slice(big, flat, (j * region_words + pad,))
  big2d = big.reshape(-1, _LANES)

  vector_mesh = plsc.VectorSubcoreMesh(
      core_axis_name="c", subcore_axis_name="s")
  scalar_mesh = plsc.ScalarSubcoreMesh(axis_name="c", num_cores=_NUM_CORES)

  tec_fn = functools.partial(
      _tec_body, q_len, v_len, dim, max_pos, region_rows)
  scs_fn = functools.partial(
      _scs_body, q_len, v_len, dim, max_pos, region_rows)

  n_rows = batch * q_len
  run = mpmd.mpmd_map(
      [(scalar_mesh, scs_fn), (vector_mesh, tec_fn)],
      out_types=[
          jax.ShapeDtypeStruct((n_rows * v_len * dim // _LANES, _LANES),
                               jnp.float32),
      ],
      scratch_types=[
          (pltpu.VMEM @ vector_mesh)((region_rows, _LANES), jnp.float32),
          pltpu.VMEM_SHARED((16 * _SCS_REGION_ROWS, _LANES), jnp.float32),
          pltpu.SemaphoreType.DMA @ vector_mesh,
          pltpu.SemaphoreType.DMA @ scalar_mesh,
      ],
  )
  out, = run(big2d)
  return out.reshape(batch, q_len, v_len, dim)


# speedup vs baseline: 1.2514x; 1.2514x over previous
import functools

import jax
import jax.numpy as jnp
from jax import lax
from jax.experimental import pallas as pl

_LANES = 128
_QB = 8


def _tc_body(q_len, v_len, dim, max_pos, b_base_words, big_ref, out_ref):
  j = pl.program_id(1)
  row_rows = v_len * dim // _LANES
  for r in range(_QB):
    q_row = j * _QB + r
    s = (max_pos - q_row) * dim
    parity = lax.rem(lax.div(s, 64), 2)
    src_row = lax.div(s + parity * b_base_words, _LANES)
    out_ref[0, r] = big_ref[pl.ds(src_row, row_rows), :]


def kernel(q, v, embeddings):
  batch, q_len = q.shape[0], q.shape[1]
  v_len = v.shape[1]
  table_rows, dim = embeddings.shape
  max_pos = (table_rows - 1) // 2

  table_words = table_rows * dim
  a_words = -(-table_words // _LANES) * _LANES
  b_base = a_words + 64
  total_words = -(-(b_base + table_words) // _LANES) * _LANES

  flat = embeddings.reshape(-1)
  big = jnp.zeros((total_words,), jnp.float32)
  big = lax.dynamic_update_slice(big, flat, (0,))
  big = lax.dynamic_update_slice(big, flat, (b_base,))
  big2d = big.reshape(-1, _LANES)
  n_big_rows = big2d.shape[0]

  row_rows = v_len * dim // _LANES
  grid = (batch, q_len // _QB)
  body = functools.partial(_tc_body, q_len, v_len, dim, max_pos, b_base)
  out = pl.pallas_call(
      body,
      grid=grid,
      in_specs=[pl.BlockSpec((n_big_rows, _LANES), lambda b, j: (0, 0))],
      out_specs=pl.BlockSpec((1, _QB, row_rows, _LANES),
                             lambda b, j: (b, j, 0, 0)),
      out_shape=jax.ShapeDtypeStruct((batch, q_len, row_rows, _LANES),
                                     jnp.float32),
  )(big2d)
  return out.reshape(batch, q_len, v_len, dim)
